# Initial kernel scaffold; baseline (speedup 1.0000x reference)
import jax
import jax.numpy as jnp
from jax.experimental import pallas as pl


def kernel(uids, gids, user_emb, item_emb, W1, b1, W2, b2):
    raise NotImplementedError("write your pallas kernel here")


# speedup vs baseline: 1.6280x; 1.6280x over previous
import functools

import jax
import jax.numpy as jnp
from jax import lax
from jax.experimental import pallas as pl
from jax.experimental.pallas import tpu as pltpu
from jax.experimental.pallas import tpu_sc as plsc

_B = 16384
_D = 128


def _build_sc_gather():
    info = plsc.get_sparse_core_info()
    nc, ns = info.num_cores, info.num_subcores
    nw = nc * ns
    b_per_w = _B // nw
    half = b_per_w // 2
    ch = 128

    mesh = plsc.VectorSubcoreMesh(core_axis_name="c", subcore_axis_name="s")

    @functools.partial(
        pl.kernel,
        out_type=(
            jax.ShapeDtypeStruct((_B, _D), jnp.float32),
            jax.ShapeDtypeStruct((_B, _D), jnp.float32),
        ),
        mesh=mesh,
        scratch_types=[
            pltpu.VMEM((b_per_w,), jnp.int32),
            pltpu.VMEM((b_per_w,), jnp.int32),
            pltpu.VMEM((half, _D), jnp.float32),
            pltpu.VMEM((half, _D), jnp.float32),
            pltpu.SemaphoreType.DMA,
            pltpu.SemaphoreType.DMA,
        ],
    )
    def gather_k(uids_hbm, gids_hbm, uemb_hbm, iemb_hbm, u_out, it_out,
                 idxu_v, idxg_v, rows_u, rows_it, sem_u, sem_it):
        wid = lax.axis_index("s") * nc + lax.axis_index("c")
        base = wid * b_per_w
        pltpu.sync_copy(uids_hbm.at[pl.ds(base, b_per_w)], idxu_v)
        pltpu.sync_copy(gids_hbm.at[pl.ds(base, b_per_w)], idxg_v)
        for h in range(b_per_w // half):
            cps = []
            for j in range(half // ch):
                off = h * half + j * ch
                cps.append(pltpu.make_async_copy(
                    uemb_hbm.at[idxu_v.at[pl.ds(off, ch)]],
                    rows_u.at[pl.ds(j * ch, ch)], sem_u))
                cps.append(pltpu.make_async_copy(
                    iemb_hbm.at[idxg_v.at[pl.ds(off, ch)]],
                    rows_it.at[pl.ds(j * ch, ch)], sem_it))
            for c in cps:
                c.start()
            for c in cps:
                c.wait()
            pltpu.sync_copy(rows_u, u_out.at[pl.ds(base + h * half, half)])
            pltpu.sync_copy(rows_it, it_out.at[pl.ds(base + h * half, half)])

    return gather_k


_sc_gather = _build_sc_gather()

_BLK = 2048


def _tc_mlp_dot(u_rows, it_rows, W1, b1, W2, b2):
    nblk = _B // _BLK

    def body(u_ref, it_ref, w1_ref, b1_ref, w2_ref, b2_ref, out_ref):
        it = it_ref[...]
        h = jnp.dot(it, w1_ref[...], preferred_element_type=jnp.float32)
        h = jnp.maximum(h + b1_ref[...], 0.0)
        h = jnp.dot(h, w2_ref[...], preferred_element_type=jnp.float32)
        h = h + b2_ref[...]
        out_ref[...] = jnp.sum(u_ref[...] * h, axis=1)[None, None, :]

    out = pl.pallas_call(
        body,
        grid=(nblk,),
        in_specs=[
            pl.BlockSpec((_BLK, _D), lambda i: (i, 0)),
            pl.BlockSpec((_BLK, _D), lambda i: (i, 0)),
            pl.BlockSpec((_D, _D), lambda i: (0, 0)),
            pl.BlockSpec((1, _D), lambda i: (0, 0)),
            pl.BlockSpec((_D, _D), lambda i: (0, 0)),
            pl.BlockSpec((1, _D), lambda i: (0, 0)),
        ],
        out_specs=pl.BlockSpec((1, 1, _BLK), lambda i: (i, 0, 0)),
        out_shape=jax.ShapeDtypeStruct((nblk, 1, _BLK), jnp.float32),
    )(u_rows, it_rows, W1, b1.reshape(1, _D), W2, b2.reshape(1, _D))
    return out.reshape(_B)


def kernel(uids, gids, user_emb, item_emb, W1, b1, W2, b2):
    uids = uids.astype(jnp.int32)
    gids = gids.astype(jnp.int32)
    u_rows, it_rows = _sc_gather(uids, gids, user_emb, item_emb)
    return _tc_mlp_dot(u_rows, it_rows, W1, b1, W2, b2)


# speedup vs baseline: 1.6358x; 1.0048x over previous
import functools

import jax
import jax.numpy as jnp
from jax import lax
from jax.experimental import pallas as pl
from jax.experimental.pallas import tpu as pltpu
from jax.experimental.pallas import tpu_sc as plsc

_B = 16384
_D = 128


def _build_sc_gather():
    info = plsc.get_sparse_core_info()
    nc, ns = info.num_cores, info.num_subcores
    nw = nc * ns
    b_per_w = _B // nw
    half = b_per_w // 2
    ch = 128

    mesh = plsc.VectorSubcoreMesh(core_axis_name="c", subcore_axis_name="s")

    @functools.partial(
        pl.kernel,
        out_type=(
            jax.ShapeDtypeStruct((_B, _D), jnp.float32),
            jax.ShapeDtypeStruct((_B, _D), jnp.float32),
        ),
        mesh=mesh,
        scratch_types=[
            pltpu.VMEM((b_per_w,), jnp.int32),
            pltpu.VMEM((b_per_w,), jnp.int32),
            pltpu.VMEM((half, _D), jnp.float32),
            pltpu.VMEM((half, _D), jnp.float32),
            pltpu.SemaphoreType.DMA,
            pltpu.SemaphoreType.DMA,
        ],
    )
    def gather_k(uids_hbm, gids_hbm, uemb_hbm, iemb_hbm, u_out, it_out,
                 idxu_v, idxg_v, rows_u, rows_it, sem_u, sem_it):
        wid = lax.axis_index("s") * nc + lax.axis_index("c")
        base = wid * b_per_w
        pltpu.sync_copy(uids_hbm.at[pl.ds(base, b_per_w)], idxu_v)
        pltpu.sync_copy(gids_hbm.at[pl.ds(base, b_per_w)], idxg_v)
        for h in range(b_per_w // half):
            cps = []
            for j in range(half // ch):
                off = h * half + j * ch
                cps.append(pltpu.make_async_copy(
                    uemb_hbm.at[idxu_v.at[pl.ds(off, ch)]],
                    rows_u.at[pl.ds(j * ch, ch)], sem_u))
                cps.append(pltpu.make_async_copy(
                    iemb_hbm.at[idxg_v.at[pl.ds(off, ch)]],
                    rows_it.at[pl.ds(j * ch, ch)], sem_it))
            for c in cps:
                c.start()
            for c in cps:
                c.wait()
            pltpu.sync_copy(rows_u, u_out.at[pl.ds(base + h * half, half)])
            pltpu.sync_copy(rows_it, it_out.at[pl.ds(base + h * half, half)])

    return gather_k


_sc_gather = _build_sc_gather()

_BLK = 2048


def _tc_mlp_dot(u_rows, it_rows, W1, b1, W2, b2):
    nblk = _B // _BLK

    def body(u_ref, it_ref, w1_ref, b1_ref, w2_ref, b2_ref, out_ref):
        it = it_ref[...].astype(jnp.bfloat16)
        h = jnp.dot(it, w1_ref[...].astype(jnp.bfloat16),
                    preferred_element_type=jnp.float32)
        h = jnp.maximum(h + b1_ref[...], 0.0)
        h = jnp.dot(h.astype(jnp.bfloat16), w2_ref[...].astype(jnp.bfloat16),
                    preferred_element_type=jnp.float32)
        h = h + b2_ref[...]
        out_ref[...] = jnp.sum(u_ref[...] * h, axis=1)[None, None, :]

    out = pl.pallas_call(
        body,
        grid=(nblk,),
        in_specs=[
            pl.BlockSpec((_BLK, _D), lambda i: (i, 0)),
            pl.BlockSpec((_BLK, _D), lambda i: (i, 0)),
            pl.BlockSpec((_D, _D), lambda i: (0, 0)),
            pl.BlockSpec((1, _D), lambda i: (0, 0)),
            pl.BlockSpec((_D, _D), lambda i: (0, 0)),
            pl.BlockSpec((1, _D), lambda i: (0, 0)),
        ],
        out_specs=pl.BlockSpec((1, 1, _BLK), lambda i: (i, 0, 0)),
        out_shape=jax.ShapeDtypeStruct((nblk, 1, _BLK), jnp.float32),
    )(u_rows, it_rows, W1, b1.reshape(1, _D), W2, b2.reshape(1, _D))
    return out.reshape(_B)


def kernel(uids, gids, user_emb, item_emb, W1, b1, W2, b2):
    uids = uids.astype(jnp.int32)
    gids = gids.astype(jnp.int32)
    u_rows, it_rows = _sc_gather(uids, gids, user_emb, item_emb)
    return _tc_mlp_dot(u_rows, it_rows, W1, b1, W2, b2)


# speedup vs baseline: 2.2496x; 1.3752x over previous
import functools

import jax
import jax.numpy as jnp
from jax import lax
from jax.experimental import pallas as pl
from jax.experimental.pallas import tpu as pltpu
from jax.experimental.pallas import tpu_sc as plsc

_B = 16384
_D = 128


def _build_sc_gather():
    info = plsc.get_sparse_core_info()
    nc, ns = info.num_cores, info.num_subcores
    nw = nc * ns
    b_per_w = _B // nw
    half = b_per_w // 2
    ch = 128

    mesh = plsc.VectorSubcoreMesh(core_axis_name="c", subcore_axis_name="s")

    @functools.partial(
        pl.kernel,
        out_type=(
            jax.ShapeDtypeStruct((_B, _D), jnp.float32),
            jax.ShapeDtypeStruct((_B, _D), jnp.float32),
        ),
        mesh=mesh,
        scratch_types=[
            pltpu.VMEM((b_per_w,), jnp.int32),
            pltpu.VMEM((b_per_w,), jnp.int32),
            pltpu.VMEM((half, _D), jnp.float32),
            pltpu.VMEM((half, _D), jnp.float32),
            pltpu.SemaphoreType.DMA,
            pltpu.SemaphoreType.DMA,
        ],
    )
    def gather_k(uids_hbm, gids_hbm, uemb_hbm, iemb_hbm, u_out, it_out,
                 idxu_v, idxg_v, rows_u, rows_it, sem_u, sem_it):
        wid = lax.axis_index("s") * nc + lax.axis_index("c")
        base = wid * b_per_w
        pltpu.sync_copy(uids_hbm.at[pl.ds(base, b_per_w)], idxu_v)
        pltpu.sync_copy(gids_hbm.at[pl.ds(base, b_per_w)], idxg_v)
        for h in range(b_per_w // half):
            cps = []
            for j in range(half // ch):
                off = h * half + j * ch
                cps.append(pltpu.make_async_copy(
                    uemb_hbm.at[idxu_v.at[pl.ds(off, ch)]],
                    rows_u.at[pl.ds(j * ch, ch)], sem_u))
                cps.append(pltpu.make_async_copy(
                    iemb_hbm.at[idxg_v.at[pl.ds(off, ch)]],
                    rows_it.at[pl.ds(j * ch, ch)], sem_it))
            for c in cps:
                c.start()
            for c in cps:
                c.wait()
            pltpu.sync_copy(rows_u, u_out.at[pl.ds(base + h * half, half)])
            pltpu.sync_copy(rows_it, it_out.at[pl.ds(base + h * half, half)])

    return gather_k


_sc_gather = _build_sc_gather()

_BLK = 2048


def _tc_mlp_dot(u_rows, it_rows, W1, b1, W2, b2):
    nblk = _B // _BLK

    def body(u_ref, it_ref, w1_ref, b1_ref, w2_ref, b2_ref, out_ref):
        it = it_ref[...]
        h = jnp.dot(it, w1_ref[...], preferred_element_type=jnp.float32)
        h = jnp.maximum(h + b1_ref[...], 0.0)
        h = jnp.dot(h, w2_ref[...], preferred_element_type=jnp.float32)
        h = h + b2_ref[...]
        p = u_ref[...] * h
        out_ref[...] = jnp.sum(p.T, axis=0)[None, None, :]

    out = pl.pallas_call(
        body,
        grid=(nblk,),
        in_specs=[
            pl.BlockSpec((_BLK, _D), lambda i: (i, 0)),
            pl.BlockSpec((_BLK, _D), lambda i: (i, 0)),
            pl.BlockSpec((_D, _D), lambda i: (0, 0)),
            pl.BlockSpec((1, _D), lambda i: (0, 0)),
            pl.BlockSpec((_D, _D), lambda i: (0, 0)),
            pl.BlockSpec((1, _D), lambda i: (0, 0)),
        ],
        out_specs=pl.BlockSpec((1, 1, _BLK), lambda i: (i, 0, 0)),
        out_shape=jax.ShapeDtypeStruct((nblk, 1, _BLK), jnp.float32),
    )(u_rows, it_rows, W1, b1.reshape(1, _D), W2, b2.reshape(1, _D))
    return out.reshape(_B)


def kernel(uids, gids, user_emb, item_emb, W1, b1, W2, b2):
    uids = uids.astype(jnp.int32)
    gids = gids.astype(jnp.int32)
    u_rows, it_rows = _sc_gather(uids, gids, user_emb, item_emb)
    return _tc_mlp_dot(u_rows, it_rows, W1, b1, W2, b2)
